# Initial kernel scaffold; baseline (speedup 1.0000x reference)
import jax
import jax.numpy as jnp
from jax.experimental import pallas as pl


def kernel(ff_path, ffpe_path, ff_node_idx, ff_hedge_idx, ffpe_node_idx, ffpe_hedge_idx, share_node_idx, share_hedge_idx, Wff1, bff1, Wff2, bff2, Wfp1, bfp1, Wfp2, bfp2, Wg_ff, bg_ff, Wg_fp, bg_fp, Wg_sh, bg_sh, Wm, bm, Wc, bc):
    raise NotImplementedError("write your pallas kernel here")


# speedup vs baseline: 1.3172x; 1.3172x over previous
import functools

import jax
import jax.numpy as jnp
from jax import lax
from jax.experimental import pallas as pl
from jax.experimental.pallas import tpu as pltpu
from jax.experimental.pallas import tpu_sc as plsc

_N = 10000
_H = 2500
_E = 160000
_N2 = 20000
_H2 = 5000
_D = 256
_CW = 128

_NC = 2
_NS = 16
_CHUNK = 128
_NCHUNK = _E // _CHUNK


def _pad16(n):
    return ((n + 15) // 16) * 16


_HP = _pad16(_H)
_HP2 = _pad16(_H2)


def _mlp_body(x_ref, w1_ref, b1_ref, w2_ref, b2_ref, o_ref):
    h = jnp.dot(x_ref[...], w1_ref[...], preferred_element_type=jnp.float32)
    h = jnp.clip(h + b1_ref[...], 0.0, 6.0)
    y = jnp.dot(h, w2_ref[...], preferred_element_type=jnp.float32)
    o_ref[...] = jnp.clip(y + b2_ref[...], 0.0, 6.0)


def _mlp(x, w1, b1, w2, b2):
    m, kin = x.shape
    bm = 1000
    return pl.pallas_call(
        _mlp_body,
        grid=(m // bm,),
        in_specs=[
            pl.BlockSpec((bm, kin), lambda i: (i, 0)),
            pl.BlockSpec((kin, _D), lambda i: (0, 0)),
            pl.BlockSpec((1, _D), lambda i: (0, 0)),
            pl.BlockSpec((_D, _D), lambda i: (0, 0)),
            pl.BlockSpec((1, _D), lambda i: (0, 0)),
        ],
        out_specs=pl.BlockSpec((bm, _D), lambda i: (i, 0)),
        out_shape=jax.ShapeDtypeStruct((m, _D), jnp.float32),
    )(x, w1, b1.reshape(1, _D), w2, b2.reshape(1, _D))


def _lin_body(x_ref, w_ref, b_ref, o_ref, *, kin, cwin):
    bmrows = x_ref.shape[1]
    y = jnp.broadcast_to(b_ref[...], (bmrows, _D))
    for k in range(kin):
        y = y + jnp.dot(x_ref[k], w_ref[k * cwin:(k + 1) * cwin, :],
                        preferred_element_type=jnp.float32)
    y = jnp.maximum(y, 0.0)
    for k in range(2):
        o_ref[k] = y[:, k * _CW:(k + 1) * _CW]


def _lin_relu(xc, w, b):
    kin, m, cwin = xc.shape
    bm = 1000
    return pl.pallas_call(
        functools.partial(_lin_body, kin=kin, cwin=cwin),
        grid=(m // bm,),
        in_specs=[
            pl.BlockSpec((kin, bm, cwin), lambda i: (0, i, 0)),
            pl.BlockSpec((_D, _D), lambda i: (0, 0)),
            pl.BlockSpec((1, _D), lambda i: (0, 0)),
        ],
        out_specs=pl.BlockSpec((2, bm, _CW), lambda i: (0, i, 0)),
        out_shape=jax.ShapeDtypeStruct((2, m, _CW), jnp.float32),
    )(xc, w, b.reshape(1, _D))


def _head_body(ff_ref, fp_ref, sht_ref, shb_ref, wm_ref, bm_ref, wc_ref,
               bc_ref, o_ref, acc, *, nb):
    i = pl.program_id(0)

    @pl.when(i == 0)
    def _():
        acc[...] = jnp.zeros_like(acc)

    sff = jnp.concatenate(
        [jnp.sum(ff_ref[k], axis=0, keepdims=True) for k in range(2)], axis=1)
    sfp = jnp.concatenate(
        [jnp.sum(fp_ref[k], axis=0, keepdims=True) for k in range(2)], axis=1)
    spt = jnp.concatenate(
        [jnp.sum(sht_ref[k], axis=0, keepdims=True) for k in range(2)], axis=1)
    spb = jnp.concatenate(
        [jnp.sum(shb_ref[k], axis=0, keepdims=True) for k in range(2)], axis=1)
    acc[...] += jnp.concatenate([sff + spt, sfp + spb], axis=1)

    @pl.when(i == nb - 1)
    def _():
        pooled = acc[...] / float(_N)
        h = jnp.dot(pooled, wm_ref[...], preferred_element_type=jnp.float32)
        h = jnp.clip(h + bm_ref[...], 0.0, 6.0)
        o_ref[...] = jnp.dot(h, wc_ref[...],
                             preferred_element_type=jnp.float32) + bc_ref[...]


def _head(ffc, fpc, shc, wm, bmv, wc, bcv):
    bm = 1000
    nb = _N // bm
    wc_pad = jnp.zeros((_D // 2, 128), jnp.float32).at[:, :4].set(wc)
    bc_pad = jnp.zeros((1, 128), jnp.float32).at[0, :4].set(bcv)
    out = pl.pallas_call(
        functools.partial(_head_body, nb=nb),
        grid=(nb,),
        in_specs=[
            pl.BlockSpec((2, bm, _CW), lambda i: (0, i, 0)),
            pl.BlockSpec((2, bm, _CW), lambda i: (0, i, 0)),
            pl.BlockSpec((2, bm, _CW), lambda i: (0, i, 0)),
            pl.BlockSpec((2, bm, _CW), lambda i: (0, i + nb, 0)),
            pl.BlockSpec((2 * _D, _D // 2), lambda i: (0, 0)),
            pl.BlockSpec((1, _D // 2), lambda i: (0, 0)),
            pl.BlockSpec((_D // 2, 128), lambda i: (0, 0)),
            pl.BlockSpec((1, 128), lambda i: (0, 0)),
        ],
        out_specs=pl.BlockSpec((1, 128), lambda i: (0, 0)),
        out_shape=jax.ShapeDtypeStruct((1, 128), jnp.float32),
        scratch_shapes=[pltpu.VMEM((1, 2 * _D), jnp.float32)],
    )(ffc, fpc, shc, shc, wm, bmv.reshape(1, _D // 2), wc_pad, bc_pad)
    return out[0, :4]


def _counts_sc(nidx, hidx, hpad, mpad, npass):
    mesh = plsc.VectorSubcoreMesh(core_axis_name="c", subcore_axis_name="s",
                                  num_cores=_NC, num_subcores=_NS)
    mpass = mpad // npass
    mb = max(hpad, mpass + 16)

    @functools.partial(
        pl.kernel,
        out_type=(jax.ShapeDtypeStruct((hpad, 16), jnp.float32),
                  jax.ShapeDtypeStruct((mpad, 16), jnp.float32)),
        mesh=mesh,
        scratch_types=[
            pltpu.VMEM((_CHUNK,), jnp.int32),
            pltpu.VMEM((_CHUNK,), jnp.int32),
            pltpu.VMEM((_CHUNK, _CW), jnp.float32),
            pltpu.VMEM((16, _CW), jnp.float32),
            pltpu.VMEM((16, 16), jnp.float32),
            pltpu.VMEM_SHARED((mb, _CW), jnp.float32),
            pltpu.SemaphoreType.DMA,
        ],
    )
    def kern(nidx_h, hidx_h, invh_h, invn_h, idx_v, idx2_v, ones_v, blk_v,
             inv16_v, acc_sh, sem):
        c = lax.axis_index("c")
        s = lax.axis_index("s")
        def of(r, _):
            for j in range(_CW // 16):
                ones_v[r, pl.ds(j * 16, 16)] = jnp.ones((16,), jnp.float32)
            return 0

        lax.fori_loop(0, _CHUNK, of, 0)

        ntc = (_NCHUNK + _NS - 1 - s) // _NS

        def do_pass(idx_h, zrows, nrows, rbase, inv_h, remap):
            for r in range(16):
                for j in range(_CW // 16):
                    blk_v[r, pl.ds(j * 16, 16)] = jnp.zeros((16,), jnp.float32)
            nzb = zrows // 16
            ntz = (nzb + _NS - 1 - s) // _NS

            def zb(i, _):
                pltpu.sync_copy(blk_v, acc_sh.at[pl.ds((s + i * _NS) * 16, 16)])
                return 0

            lax.fori_loop(0, ntz, zb, 0)
            plsc.subcore_barrier()

            def cb(t, _):
                base = (s + t * _NS) * _CHUNK
                pltpu.sync_copy(idx_h.at[pl.ds(base, _CHUNK)], idx_v)
                if remap:
                    for j in range(_CHUNK // 16):
                        v = idx_v[pl.ds(j * 16, 16)] - rbase
                        ok = (v >= 0) & (v < nrows)
                        idx2_v[pl.ds(j * 16, 16)] = jnp.where(ok, v, nrows)
                    pltpu.sync_copy(ones_v, acc_sh.at[idx2_v], add=True)
                else:
                    pltpu.sync_copy(ones_v, acc_sh.at[idx_v], add=True)
                return 0

            lax.fori_loop(0, ntc, cb, 0)
            plsc.subcore_barrier()

            nnb = nrows // 16
            ntn = (nnb + _NS - 1 - s) // _NS

            def nb(i, _):
                r = (s + i * _NS) * 16
                pltpu.sync_copy(acc_sh.at[pl.ds(r, 16)], blk_v)
                for ri in range(16):
                    v = blk_v[ri, pl.ds(0, 16)]
                    inv16_v[ri, :] = 1.0 / jnp.maximum(v, 1.0)
                pltpu.sync_copy(inv16_v, inv_h.at[pl.ds(rbase + r, 16)])
                return 0

            lax.fori_loop(0, ntn, nb, 0)
            plsc.subcore_barrier()

        @pl.when(c == 0)
        def _():
            do_pass(hidx_h, hpad, hpad, 0, invh_h, False)
            for _p in range(npass - 1):
                plsc.subcore_barrier()
                plsc.subcore_barrier()
                plsc.subcore_barrier()

        @pl.when(c == 1)
        def _():
            for p in range(npass):
                do_pass(nidx_h, mpass + 16, mpass, p * mpass, invn_h, True)

    return kern(nidx, hidx)


def _conv_sc(nidx, hidx, xc, invh, invn, hpad, m, nrp):
    mesh = plsc.VectorSubcoreMesh(core_axis_name="c", subcore_axis_name="s",
                                  num_cores=_NC, num_subcores=_NS)
    mpass = m // nrp

    @functools.partial(
        pl.kernel,
        out_type=(jax.ShapeDtypeStruct((2, m, _CW), jnp.float32),
                  jax.ShapeDtypeStruct((2, hpad, _CW), jnp.float32)),
        mesh=mesh,
        scratch_types=[
            pltpu.VMEM((_CHUNK,), jnp.int32),
            pltpu.VMEM((_CHUNK,), jnp.int32),
            pltpu.VMEM((_CHUNK,), jnp.int32),
            pltpu.VMEM((_CHUNK, _CW), jnp.float32),
            pltpu.VMEM((16, _CW), jnp.float32),
            pltpu.VMEM((16, 16), jnp.float32),
            pltpu.VMEM_SHARED((max(hpad, mpass + 16), _CW), jnp.float32),
            pltpu.SemaphoreType.DMA,
        ],
    )
    def kern(nidx_h, hidx_h, xc_h, invh_h, invn_h, out_h, he_h,
             ia_v, ib_v, ic_v, rows_v, nb_v, iv_v, acc_sh, sem):
        he_sh = acc_sh
        out_sh = acc_sh
        c = lax.axis_index("c")
        s = lax.axis_index("s")
        nblkh = hpad // 16
        nth = (nblkh + _NS - 1 - s) // _NS
        nblko = (mpass + 16) // 16
        nto = (nblko + _NS - 1 - s) // _NS
        nblkm = mpass // 16
        ntm = (nblkm + _NS - 1 - s) // _NS
        ntc = (_NCHUNK + _NS - 1 - s) // _NS

        def zero_nb():
            for r in range(16):
                for j in range(_CW // 16):
                    nb_v[r, pl.ds(j * 16, 16)] = jnp.zeros((16,), jnp.float32)

        def scale_nb():
            for ri in range(16):
                f = iv_v[ri, :]
                for j in range(_CW // 16):
                    nb_v[ri, pl.ds(j * 16, 16)] = nb_v[ri, pl.ds(j * 16, 16)] * f

        def run_core(ci):
            zero_nb()

            def zb(i, _):
                pltpu.sync_copy(nb_v, he_sh.at[pl.ds((s + i * _NS) * 16, 16)])
                return 0

            lax.fori_loop(0, nth, zb, 0)
            plsc.subcore_barrier()

            def sa(t, _):
                base = (s + t * _NS) * _CHUNK
                pltpu.sync_copy(nidx_h.at[pl.ds(base, _CHUNK)], ia_v)
                pltpu.sync_copy(hidx_h.at[pl.ds(base, _CHUNK)], ib_v)
                pltpu.async_copy(xc_h.at[ci].at[ia_v], rows_v, sem).wait()
                pltpu.sync_copy(rows_v, he_sh.at[ib_v], add=True)
                return 0

            lax.fori_loop(0, ntc, sa, 0)
            plsc.subcore_barrier()

            def hn(i, _):
                r = (s + i * _NS) * 16
                pltpu.sync_copy(he_sh.at[pl.ds(r, 16)], nb_v)
                pltpu.sync_copy(invh_h.at[pl.ds(r, 16)], iv_v)
                scale_nb()
                pltpu.sync_copy(nb_v, he_h.at[ci].at[pl.ds(r, 16)])
                return 0

            lax.fori_loop(0, nth, hn, 0)
            plsc.subcore_barrier()

            for rp in range(nrp):
                nbase = rp * mpass
                zero_nb()

                def zb2(i, _):
                    pltpu.sync_copy(
                        nb_v, out_sh.at[pl.ds((s + i * _NS) * 16, 16)])
                    return 0

                lax.fori_loop(0, nto, zb2, 0)
                plsc.subcore_barrier()

                def sb(t, _):
                    base = (s + t * _NS) * _CHUNK
                    pltpu.sync_copy(nidx_h.at[pl.ds(base, _CHUNK)], ia_v)
                    pltpu.sync_copy(hidx_h.at[pl.ds(base, _CHUNK)], ib_v)
                    pltpu.async_copy(he_h.at[ci].at[ib_v], rows_v, sem).wait()
                    if nrp == 1:
                        pltpu.sync_copy(rows_v, out_sh.at[ia_v], add=True)
                    else:
                        for j in range(_CHUNK // 16):
                            v = ia_v[pl.ds(j * 16, 16)] - nbase
                            ok = (v >= 0) & (v < mpass)
                            ic_v[pl.ds(j * 16, 16)] = jnp.where(ok, v, mpass)
                        pltpu.sync_copy(rows_v, out_sh.at[ic_v], add=True)
                    return 0

                lax.fori_loop(0, ntc, sb, 0)
                plsc.subcore_barrier()

                def on(i, _):
                    r = (s + i * _NS) * 16
                    pltpu.sync_copy(out_sh.at[pl.ds(r, 16)], nb_v)
                    pltpu.sync_copy(invn_h.at[pl.ds(nbase + r, 16)], iv_v)
                    scale_nb()
                    pltpu.sync_copy(nb_v, out_h.at[ci].at[pl.ds(nbase + r, 16)])
                    return 0

                lax.fori_loop(0, ntm, on, 0)
                plsc.subcore_barrier()

        @pl.when(c == 0)
        def _():
            run_core(0)

        @pl.when(c == 1)
        def _():
            run_core(1)

    out, _ = kern(nidx, hidx, xc, invh, invn)
    return out


def _tie(dep, *xs):
    z = (jnp.sum(dep) * 0.0).astype(jnp.int32)
    return tuple(x + z for x in xs)


def kernel(ff_path, ffpe_path, ff_node_idx, ff_hedge_idx, ffpe_node_idx,
           ffpe_hedge_idx, share_node_idx, share_hedge_idx, Wff1, bff1, Wff2,
           bff2, Wfp1, bfp1, Wfp2, bfp2, Wg_ff, bg_ff, Wg_fp, bg_fp, Wg_sh,
           bg_sh, Wm, bm, Wc, bc):
    ff = _mlp(ff_path[0], Wff1, bff1, Wff2, bff2)
    fp = _mlp(ffpe_path[0], Wfp1, bfp1, Wfp2, bfp2)
    p0 = jnp.concatenate([ff, fp], axis=0)

    invh_ff, invn_ff = _counts_sc(ff_node_idx, ff_hedge_idx, _HP, _N, 1)
    a, b = _tie(invn_ff[:16], ffpe_node_idx, ffpe_hedge_idx)
    invh_fp, invn_fp = _counts_sc(a, b, _HP, _N, 1)
    a, b = _tie(invn_fp[:16], share_node_idx, share_hedge_idx)
    invh_sh, invn_sh = _counts_sc(a, b, _HP2, _N2, 2)

    tok = invn_sh[:16]
    ffc = ff.reshape(1, _N, _D)
    fpc = fp.reshape(1, _N, _D)
    shc = p0.reshape(1, _N2, _D)
    for l in range(3):
        xa = _lin_relu(ffc, Wg_ff[l], bg_ff[l])
        a, b = _tie(tok, ff_node_idx, ff_hedge_idx)
        ffc = _conv_sc(a, b, xa, invh_ff, invn_ff, _HP, _N, 1)
        tok = ffc[0, :16]
        xb = _lin_relu(fpc, Wg_fp[l], bg_fp[l])
        a, b = _tie(tok, ffpe_node_idx, ffpe_hedge_idx)
        fpc = _conv_sc(a, b, xb, invh_fp, invn_fp, _HP, _N, 1)
        tok = fpc[0, :16]
        xs = _lin_relu(shc, Wg_sh[l], bg_sh[l])
        a, b = _tie(tok, share_node_idx, share_hedge_idx)
        shc = _conv_sc(a, b, xs, invh_sh, invn_sh, _HP2, _N2, 2)
        tok = shc[0, :16]

    return _head(ffc, fpc, shc, Wm, bm, Wc, bc)


# speedup vs baseline: 2.0461x; 1.5534x over previous
import functools

import jax
import jax.numpy as jnp
from jax import lax
from jax.experimental import pallas as pl
from jax.experimental.pallas import tpu as pltpu
from jax.experimental.pallas import tpu_sc as plsc

_N = 10000
_H = 2500
_E = 160000
_N2 = 20000
_H2 = 5000
_D = 256
_CW = 128

_NC = 2
_NS = 16
_CHUNK = 128
_NCHUNK = _E // _CHUNK


def _pad16(n):
    return ((n + 15) // 16) * 16


_HP = _pad16(_H)
_HP2 = _pad16(_H2)


def _mlp_body(x_ref, w1_ref, b1_ref, w2_ref, b2_ref, o_ref):
    h = jnp.dot(x_ref[...], w1_ref[...], preferred_element_type=jnp.float32)
    h = jnp.clip(h + b1_ref[...], 0.0, 6.0)
    y = jnp.dot(h, w2_ref[...], preferred_element_type=jnp.float32)
    o_ref[...] = jnp.clip(y + b2_ref[...], 0.0, 6.0)


def _mlp(x, w1, b1, w2, b2):
    m, kin = x.shape
    bm = 1000
    return pl.pallas_call(
        _mlp_body,
        grid=(m // bm,),
        in_specs=[
            pl.BlockSpec((bm, kin), lambda i: (i, 0)),
            pl.BlockSpec((kin, _D), lambda i: (0, 0)),
            pl.BlockSpec((1, _D), lambda i: (0, 0)),
            pl.BlockSpec((_D, _D), lambda i: (0, 0)),
            pl.BlockSpec((1, _D), lambda i: (0, 0)),
        ],
        out_specs=pl.BlockSpec((bm, _D), lambda i: (i, 0)),
        out_shape=jax.ShapeDtypeStruct((m, _D), jnp.float32),
    )(x, w1, b1.reshape(1, _D), w2, b2.reshape(1, _D))


def _lin_body(x_ref, w_ref, b_ref, o_ref, *, kin, cwin):
    bmrows = x_ref.shape[1]
    y = jnp.broadcast_to(b_ref[...], (bmrows, _D))
    for k in range(kin):
        y = y + jnp.dot(x_ref[k], w_ref[k * cwin:(k + 1) * cwin, :],
                        preferred_element_type=jnp.float32)
    y = jnp.maximum(y, 0.0)
    for k in range(2):
        o_ref[k] = y[:, k * _CW:(k + 1) * _CW]


def _lin_relu(xc, w, b):
    kin, m, cwin = xc.shape
    bm = 1000
    return pl.pallas_call(
        functools.partial(_lin_body, kin=kin, cwin=cwin),
        grid=(m // bm,),
        in_specs=[
            pl.BlockSpec((kin, bm, cwin), lambda i: (0, i, 0)),
            pl.BlockSpec((_D, _D), lambda i: (0, 0)),
            pl.BlockSpec((1, _D), lambda i: (0, 0)),
        ],
        out_specs=pl.BlockSpec((2, bm, _CW), lambda i: (0, i, 0)),
        out_shape=jax.ShapeDtypeStruct((2, m, _CW), jnp.float32),
    )(xc, w, b.reshape(1, _D))


def _head_body(ff_ref, fp_ref, sht_ref, shb_ref, wm_ref, bm_ref, wc_ref,
               bc_ref, o_ref, acc, *, nb):
    i = pl.program_id(0)

    @pl.when(i == 0)
    def _():
        acc[...] = jnp.zeros_like(acc)

    sff = jnp.concatenate(
        [jnp.sum(ff_ref[k], axis=0, keepdims=True) for k in range(2)], axis=1)
    sfp = jnp.concatenate(
        [jnp.sum(fp_ref[k], axis=0, keepdims=True) for k in range(2)], axis=1)
    spt = jnp.concatenate(
        [jnp.sum(sht_ref[k], axis=0, keepdims=True) for k in range(2)], axis=1)
    spb = jnp.concatenate(
        [jnp.sum(shb_ref[k], axis=0, keepdims=True) for k in range(2)], axis=1)
    acc[...] += jnp.concatenate([sff + spt, sfp + spb], axis=1)

    @pl.when(i == nb - 1)
    def _():
        pooled = acc[...] / float(_N)
        h = jnp.dot(pooled, wm_ref[...], preferred_element_type=jnp.float32)
        h = jnp.clip(h + bm_ref[...], 0.0, 6.0)
        o_ref[...] = jnp.dot(h, wc_ref[...],
                             preferred_element_type=jnp.float32) + bc_ref[...]


def _head(ffc, fpc, shc, wm, bmv, wc, bcv):
    bm = 1000
    nb = _N // bm
    wc_pad = jnp.zeros((_D // 2, 128), jnp.float32).at[:, :4].set(wc)
    bc_pad = jnp.zeros((1, 128), jnp.float32).at[0, :4].set(bcv)
    out = pl.pallas_call(
        functools.partial(_head_body, nb=nb),
        grid=(nb,),
        in_specs=[
            pl.BlockSpec((2, bm, _CW), lambda i: (0, i, 0)),
            pl.BlockSpec((2, bm, _CW), lambda i: (0, i, 0)),
            pl.BlockSpec((2, bm, _CW), lambda i: (0, i, 0)),
            pl.BlockSpec((2, bm, _CW), lambda i: (0, i + nb, 0)),
            pl.BlockSpec((2 * _D, _D // 2), lambda i: (0, 0)),
            pl.BlockSpec((1, _D // 2), lambda i: (0, 0)),
            pl.BlockSpec((_D // 2, 128), lambda i: (0, 0)),
            pl.BlockSpec((1, 128), lambda i: (0, 0)),
        ],
        out_specs=pl.BlockSpec((1, 128), lambda i: (0, 0)),
        out_shape=jax.ShapeDtypeStruct((1, 128), jnp.float32),
        scratch_shapes=[pltpu.VMEM((1, 2 * _D), jnp.float32)],
    )(ffc, fpc, shc, shc, wm, bmv.reshape(1, _D // 2), wc_pad, bc_pad)
    return out[0, :4]


def _counts_sc(nidx, hidx, hpad, mpad, npass):
    mesh = plsc.VectorSubcoreMesh(core_axis_name="c", subcore_axis_name="s",
                                  num_cores=_NC, num_subcores=_NS)
    mpass = mpad // npass
    mb = max(hpad, mpass + 16)

    @functools.partial(
        pl.kernel,
        out_type=(jax.ShapeDtypeStruct((hpad, 16), jnp.float32),
                  jax.ShapeDtypeStruct((mpad, 16), jnp.float32)),
        mesh=mesh,
        scratch_types=[
            pltpu.VMEM((_CHUNK,), jnp.int32),
            pltpu.VMEM((_CHUNK,), jnp.int32),
            pltpu.VMEM((_CHUNK, _CW), jnp.float32),
            pltpu.VMEM((16, _CW), jnp.float32),
            pltpu.VMEM((16, 16), jnp.float32),
            pltpu.VMEM_SHARED((mb, _CW), jnp.float32),
            pltpu.SemaphoreType.DMA,
        ],
    )
    def kern(nidx_h, hidx_h, invh_h, invn_h, idx_v, idx2_v, ones_v, blk_v,
             inv16_v, acc_sh, sem):
        c = lax.axis_index("c")
        s = lax.axis_index("s")
        def of(r, _):
            for j in range(_CW // 16):
                ones_v[r, pl.ds(j * 16, 16)] = jnp.ones((16,), jnp.float32)
            return 0

        lax.fori_loop(0, _CHUNK, of, 0)

        ntc = (_NCHUNK + _NS - 1 - s) // _NS

        def do_pass(idx_h, zrows, nrows, rbase, inv_h, remap):
            for r in range(16):
                for j in range(_CW // 16):
                    blk_v[r, pl.ds(j * 16, 16)] = jnp.zeros((16,), jnp.float32)
            nzb = zrows // 16
            ntz = (nzb + _NS - 1 - s) // _NS

            def zb(i, _):
                pltpu.sync_copy(blk_v, acc_sh.at[pl.ds((s + i * _NS) * 16, 16)])
                return 0

            lax.fori_loop(0, ntz, zb, 0)
            plsc.subcore_barrier()

            def cb(t, _):
                base = (s + t * _NS) * _CHUNK
                pltpu.sync_copy(idx_h.at[pl.ds(base, _CHUNK)], idx_v)
                if remap:
                    for j in range(_CHUNK // 16):
                        v = idx_v[pl.ds(j * 16, 16)] - rbase
                        ok = (v >= 0) & (v < nrows)
                        idx2_v[pl.ds(j * 16, 16)] = jnp.where(ok, v, nrows)
                    pltpu.sync_copy(ones_v, acc_sh.at[idx2_v], add=True)
                else:
                    pltpu.sync_copy(ones_v, acc_sh.at[idx_v], add=True)
                return 0

            lax.fori_loop(0, ntc, cb, 0)
            plsc.subcore_barrier()

            nnb = nrows // 16
            ntn = (nnb + _NS - 1 - s) // _NS

            def nb(i, _):
                r = (s + i * _NS) * 16
                pltpu.sync_copy(acc_sh.at[pl.ds(r, 16)], blk_v)
                for ri in range(16):
                    v = blk_v[ri, pl.ds(0, 16)]
                    inv16_v[ri, :] = jnp.where(
                        v > 0.5, 1.0 / jnp.maximum(v, 1.0), 0.0)
                pltpu.sync_copy(inv16_v, inv_h.at[pl.ds(rbase + r, 16)])
                return 0

            lax.fori_loop(0, ntn, nb, 0)
            plsc.subcore_barrier()

        @pl.when(c == 0)
        def _():
            do_pass(hidx_h, hpad, hpad, 0, invh_h, False)
            for _p in range(npass - 1):
                plsc.subcore_barrier()
                plsc.subcore_barrier()
                plsc.subcore_barrier()

        @pl.when(c == 1)
        def _():
            for p in range(npass):
                do_pass(nidx_h, mpass + 16, mpass, p * mpass, invn_h, True)

    return kern(nidx, hidx)


def _conv_sc(nidx, hidx, xc, invh, invn, hpad, m, nrp):
    mesh = plsc.VectorSubcoreMesh(core_axis_name="c", subcore_axis_name="s",
                                  num_cores=_NC, num_subcores=_NS)
    mpass = m // nrp

    @functools.partial(
        pl.kernel,
        out_type=(jax.ShapeDtypeStruct((2, m, _CW), jnp.float32),
                  jax.ShapeDtypeStruct((2, hpad, _CW), jnp.float32)),
        mesh=mesh,
        scratch_types=[
            pltpu.VMEM((_CHUNK,), jnp.int32),
            pltpu.VMEM((_CHUNK,), jnp.int32),
            pltpu.VMEM((_CHUNK,), jnp.int32),
            pltpu.VMEM((_CHUNK,), jnp.int32),
            pltpu.VMEM((_CHUNK,), jnp.int32),
            pltpu.VMEM((_CHUNK, _CW), jnp.float32),
            pltpu.VMEM((_CHUNK, _CW), jnp.float32),
            pltpu.VMEM((16, _CW), jnp.float32),
            pltpu.VMEM((16, 16), jnp.float32),
            pltpu.VMEM_SHARED((max(hpad, mpass + 16), _CW), jnp.float32),
            pltpu.SemaphoreType.DMA,
            pltpu.SemaphoreType.DMA,
        ],
    )
    def kern(nidx_h, hidx_h, xc_h, invh_h, invn_h, out_h, he_h,
             ia0, ib0, ia1, ib1, ic_v, rows0, rows1,
             nb_v, iv_v, acc_sh, sg0, sg1):
        c = lax.axis_index("c")
        s = lax.axis_index("s")
        nblkh = hpad // 16
        nth = (nblkh + _NS - 1 - s) // _NS
        nblko = (mpass + 16) // 16
        nto = (nblko + _NS - 1 - s) // _NS
        nblkm = mpass // 16
        ntm = (nblkm + _NS - 1 - s) // _NS
        nt = (_NCHUNK + _NS - 1 - s) // _NS

        def load_pair(tt, iaB, ibB):
            base = (s + tt * _NS) * _CHUNK
            pltpu.sync_copy(nidx_h.at[pl.ds(base, _CHUNK)], iaB)
            pltpu.sync_copy(hidx_h.at[pl.ds(base, _CHUNK)], ibB)

        def zero_nb():
            for r in range(16):
                for j in range(_CW // 16):
                    nb_v[r, pl.ds(j * 16, 16)] = jnp.zeros((16,), jnp.float32)

        def scale_nb():
            for ri in range(16):
                f = iv_v[ri, :]
                for j in range(_CW // 16):
                    nb_v[ri, pl.ds(j * 16, 16)] = nb_v[ri, pl.ds(j * 16, 16)] * f

        def run_core(ci):
            src_a = xc_h.at[ci]
            dummy_a = src_a.at[pl.ds(0, _CHUNK)]
            zero_nb()

            def zb(i, _):
                pltpu.sync_copy(nb_v, acc_sh.at[pl.ds((s + i * _NS) * 16, 16)])
                return 0

            lax.fori_loop(0, nth, zb, 0)
            plsc.subcore_barrier()

            load_pair(0, ia0, ib0)
            pltpu.async_copy(src_a.at[ia0], rows0, sg0)
            load_pair(1, ia1, ib1)
            pltpu.async_copy(src_a.at[ia1], rows1, sg1)

            def slot_a(tt, iaB, ibB, rbuf, sem):
                @pl.when(tt < nt)
                def _():
                    pltpu.make_async_copy(dummy_a, rbuf, sem).wait()
                    pltpu.sync_copy(rbuf, acc_sh.at[ibB], add=True)

                    @pl.when(tt + 2 < nt)
                    def _():
                        load_pair(tt + 2, iaB, ibB)
                        pltpu.async_copy(src_a.at[iaB], rbuf, sem)

            def sa(i, _):
                slot_a(2 * i, ia0, ib0, rows0, sg0)
                slot_a(2 * i + 1, ia1, ib1, rows1, sg1)
                return 0

            lax.fori_loop(0, 40, sa, 0)
            plsc.subcore_barrier()

            def hn(i, _):
                r = (s + i * _NS) * 16
                pltpu.sync_copy(acc_sh.at[pl.ds(r, 16)], nb_v)
                pltpu.sync_copy(invh_h.at[pl.ds(r, 16)], iv_v)
                scale_nb()
                pltpu.sync_copy(nb_v, he_h.at[ci].at[pl.ds(r, 16)])
                return 0

            lax.fori_loop(0, nth, hn, 0)
            plsc.subcore_barrier()

            src_b = he_h.at[ci]
            dummy_b = src_b.at[pl.ds(0, _CHUNK)]
            for rp in range(nrp):
                nbase = rp * mpass
                zero_nb()

                def zb2(i, _):
                    pltpu.sync_copy(
                        nb_v, acc_sh.at[pl.ds((s + i * _NS) * 16, 16)])
                    return 0

                lax.fori_loop(0, nto, zb2, 0)
                plsc.subcore_barrier()

                def remap(iaB):
                    if nrp == 1:
                        return iaB
                    for j in range(_CHUNK // 16):
                        v = iaB[pl.ds(j * 16, 16)] - nbase
                        ok = (v >= 0) & (v < mpass)
                        ic_v[pl.ds(j * 16, 16)] = jnp.where(ok, v, mpass)
                    return ic_v

                load_pair(0, ia0, ib0)
                pltpu.async_copy(src_b.at[ib0], rows0, sg0)
                load_pair(1, ia1, ib1)
                pltpu.async_copy(src_b.at[ib1], rows1, sg1)

                def slot_b(tt, iaB, ibB, rbuf, sem):
                    @pl.when(tt < nt)
                    def _():
                        pltpu.make_async_copy(dummy_b, rbuf, sem).wait()
                        pltpu.sync_copy(rbuf, acc_sh.at[remap(iaB)], add=True)

                        @pl.when(tt + 2 < nt)
                        def _():
                            load_pair(tt + 2, iaB, ibB)
                            pltpu.async_copy(src_b.at[ibB], rbuf, sem)

                def sb(i, _):
                    slot_b(2 * i, ia0, ib0, rows0, sg0)
                    slot_b(2 * i + 1, ia1, ib1, rows1, sg1)
                    return 0

                lax.fori_loop(0, 40, sb, 0)
                plsc.subcore_barrier()

                def on(i, _):
                    r = (s + i * _NS) * 16
                    pltpu.sync_copy(acc_sh.at[pl.ds(r, 16)], nb_v)
                    pltpu.sync_copy(invn_h.at[pl.ds(nbase + r, 16)], iv_v)
                    scale_nb()
                    pltpu.sync_copy(nb_v, out_h.at[ci].at[pl.ds(nbase + r, 16)])
                    return 0

                lax.fori_loop(0, ntm, on, 0)
                plsc.subcore_barrier()

        @pl.when(c == 0)
        def _():
            run_core(0)

        @pl.when(c == 1)
        def _():
            run_core(1)

    out, _ = kern(nidx, hidx, xc, invh, invn)
    return out


def _tie(dep, *xs):
    z = (jnp.sum(dep) * 0.0).astype(jnp.int32)
    return tuple(x + z for x in xs)


def kernel(ff_path, ffpe_path, ff_node_idx, ff_hedge_idx, ffpe_node_idx,
           ffpe_hedge_idx, share_node_idx, share_hedge_idx, Wff1, bff1, Wff2,
           bff2, Wfp1, bfp1, Wfp2, bfp2, Wg_ff, bg_ff, Wg_fp, bg_fp, Wg_sh,
           bg_sh, Wm, bm, Wc, bc):
    ff = _mlp(ff_path[0], Wff1, bff1, Wff2, bff2)
    fp = _mlp(ffpe_path[0], Wfp1, bfp1, Wfp2, bfp2)
    p0 = jnp.concatenate([ff, fp], axis=0)

    invh_ff, invn_ff = _counts_sc(ff_node_idx, ff_hedge_idx, _HP, _N, 1)
    a, b = _tie(invn_ff[:16], ffpe_node_idx, ffpe_hedge_idx)
    invh_fp, invn_fp = _counts_sc(a, b, _HP, _N, 1)
    a, b = _tie(invn_fp[:16], share_node_idx, share_hedge_idx)
    invh_sh, invn_sh = _counts_sc(a, b, _HP2, _N2, 2)

    tok = invn_sh[:16]
    ffc = ff.reshape(1, _N, _D)
    fpc = fp.reshape(1, _N, _D)
    shc = p0.reshape(1, _N2, _D)
    for l in range(3):
        xa = _lin_relu(ffc, Wg_ff[l], bg_ff[l])
        a, b = _tie(tok, ff_node_idx, ff_hedge_idx)
        ffc = _conv_sc(a, b, xa, invh_ff, invn_ff, _HP, _N, 1)
        tok = ffc[0, :16]
        xb = _lin_relu(fpc, Wg_fp[l], bg_fp[l])
        a, b = _tie(tok, ffpe_node_idx, ffpe_hedge_idx)
        fpc = _conv_sc(a, b, xb, invh_fp, invn_fp, _HP, _N, 1)
        tok = fpc[0, :16]
        xs = _lin_relu(shc, Wg_sh[l], bg_sh[l])
        a, b = _tie(tok, share_node_idx, share_hedge_idx)
        shc = _conv_sc(a, b, xs, invh_sh, invn_sh, _HP2, _N2, 2)
        tok = shc[0, :16]

    return _head(ffc, fpc, shc, Wm, bm, Wc, bc)
